# Initial kernel scaffold; baseline (speedup 1.0000x reference)
import jax
import jax.numpy as jnp
from jax.experimental import pallas as pl


def kernel(h_V, h_E, src_idx, batch_id, dst_idx, W1, b1, W2, b2, W3, b3, Wi, bi, Wo, bo, ln1_g, ln1_b, ln2_g, ln2_b):
    raise NotImplementedError("write your pallas kernel here")


# speedup vs baseline: 3.0403x; 3.0403x over previous
import functools

import jax
import jax.numpy as jnp
from jax import lax
from jax.experimental import pallas as pl
from jax.experimental.pallas import tpu as pltpu
from jax.experimental.pallas import tpu_sc as plsc

N = 10000
E = 160000
D = 256
DH = D // 2
FF = 4 * D
SCALE = 30.0

NC = 2
NS = 16
NW = NC * NS
EPW = E // NW
GCHUNK = 128
GFULL = EPW // GCHUNK
GTAIL = EPW - GFULL * GCHUNK
EPS = E // NS
SCHUNK = 80
SCHUNKS = EPS // SCHUNK
TASK = 80
NTASK = N // TASK

@functools.cache
def _mesh():
    return plsc.VectorSubcoreMesh(core_axis_name="c", subcore_axis_name="s",
                                  num_cores=NC, num_subcores=NS)


def _gelu(x):
    return 0.5 * x * (1.0 + lax.erf(x * 0.7071067811865476))


def _node_proj_body(hv_ref, wac_ref, a_ref, c_ref):
    y = jnp.dot(hv_ref[...], wac_ref[...], preferred_element_type=jnp.float32)
    a_ref[...] = y[:, :D]
    c_ref[...] = y[:, D:]


def _node_proj(h_V, w_ac, blk=1000):
    grid = (N // blk,)
    return pl.pallas_call(
        _node_proj_body,
        grid=grid,
        in_specs=[
            pl.BlockSpec((blk, D), lambda i: (i, 0)),
            pl.BlockSpec((D, 2 * D), lambda i: (0, 0)),
        ],
        out_specs=[
            pl.BlockSpec((blk, D), lambda i: (i, 0)),
            pl.BlockSpec((blk, D), lambda i: (i, 0)),
        ],
        out_shape=[
            jax.ShapeDtypeStruct((N, D), jnp.float32),
            jax.ShapeDtypeStruct((N, D), jnp.float32),
        ],
    )(h_V, w_ac)


def _gather_body(a_hbm, c_hbm, src_hbm, dst_hbm, g1_hbm, g2_hbm,
                 sidx_v, didx_v, bufa, bufc, sem):
    c = lax.axis_index("c")
    s = lax.axis_index("s")
    wid = s * NC + c
    base = wid * EPW
    pltpu.sync_copy(src_hbm.at[pl.ds(base, EPW)], sidx_v)
    pltpu.sync_copy(dst_hbm.at[pl.ds(base, EPW)], didx_v)

    def chunk(j, carry):
        off = j * GCHUNK
        pltpu.async_copy(a_hbm.at[sidx_v.at[pl.ds(off, GCHUNK)]], bufa, sem).wait()
        pltpu.sync_copy(bufa, g1_hbm.at[pl.ds(base + off, GCHUNK)])
        pltpu.async_copy(c_hbm.at[didx_v.at[pl.ds(off, GCHUNK)]], bufc, sem).wait()
        pltpu.sync_copy(bufc, g2_hbm.at[pl.ds(base + off, GCHUNK)])
        return carry

    lax.fori_loop(0, GFULL, chunk, 0)
    toff = GFULL * GCHUNK
    ta = bufa.at[pl.ds(0, GTAIL)]
    tc = bufc.at[pl.ds(0, GTAIL)]
    pltpu.async_copy(a_hbm.at[sidx_v.at[pl.ds(toff, GTAIL)]], ta, sem).wait()
    pltpu.sync_copy(ta, g1_hbm.at[pl.ds(base + toff, GTAIL)])
    pltpu.async_copy(c_hbm.at[didx_v.at[pl.ds(toff, GTAIL)]], tc, sem).wait()
    pltpu.sync_copy(tc, g2_hbm.at[pl.ds(base + toff, GTAIL)])


def _sc_gather(a, c, src_idx, dst_idx):
    f = pl.kernel(
        _gather_body,
        out_type=[
            jax.ShapeDtypeStruct((E, D), jnp.float32),
            jax.ShapeDtypeStruct((E, D), jnp.float32),
        ],
        mesh=_mesh(),
        scratch_types=[
            pltpu.VMEM((EPW,), jnp.int32),
            pltpu.VMEM((EPW,), jnp.int32),
            pltpu.VMEM((GCHUNK, D), jnp.float32),
            pltpu.VMEM((GCHUNK, D), jnp.float32),
            pltpu.SemaphoreType.DMA,
        ],
    )
    return f(a, c, src_idx, dst_idx)


def _edge_mlp_body(he_ref, g1_ref, g2_ref, w1b_ref, b1_ref, w2_ref, b2_ref,
                   w3_ref, b3_ref, m_ref):
    x = (g1_ref[...] + g2_ref[...] + b1_ref[...]
         + jnp.dot(he_ref[...], w1b_ref[...], preferred_element_type=jnp.float32))
    x = _gelu(x)
    y = _gelu(jnp.dot(x, w2_ref[...], preferred_element_type=jnp.float32)
              + b2_ref[...])
    m = jnp.dot(y, w3_ref[...], preferred_element_type=jnp.float32) + b3_ref[...]
    m_ref[0] = m[:, :DH]
    m_ref[1] = m[:, DH:]


def _edge_mlp(h_E, g1, g2, w1b, b1, w2, b2, w3, b3, blk=1000):
    grid = (E // blk,)
    full = lambda r, c: pl.BlockSpec((r, c), lambda i: (0, 0))
    row = lambda c: pl.BlockSpec((blk, c), lambda i: (i, 0))
    return pl.pallas_call(
        _edge_mlp_body,
        grid=grid,
        in_specs=[
            row(D), row(D), row(D),
            full(D, D),
            pl.BlockSpec((D,), lambda i: (0,)),
            full(D, D),
            pl.BlockSpec((D,), lambda i: (0,)),
            full(D, D),
            pl.BlockSpec((D,), lambda i: (0,)),
        ],
        out_specs=pl.BlockSpec((2, blk, DH), lambda i: (0, i, 0)),
        out_shape=jax.ShapeDtypeStruct((2, E, DH), jnp.float32),
    )(h_E, g1, g2, w1b, b1, w2, b2, w3, b3)


def _scatter_body(m_hbm, src3d_hbm, zsum_hbm, sout_hbm,
                  idx_v, data_v, task_v, shared_sum, sem):
    c = lax.axis_index("c")
    s = lax.axis_index("s")

    pltpu.sync_copy(zsum_hbm, task_v)
    pltpu.sync_copy(src3d_hbm.at[s], idx_v)

    def init_loop(k, carry):
        pltpu.sync_copy(task_v, shared_sum.at[pl.ds((s + k * NS) * TASK, TASK)])
        return carry

    lax.fori_loop(0, NTASK // NS, init_loop, 0)

    @pl.when(s < NTASK % NS)
    def _():
        pltpu.sync_copy(task_v,
                        shared_sum.at[pl.ds(((NTASK // NS) * NS + s) * TASK, TASK)])

    plsc.subcore_barrier()

    def chunk(j, carry):
        ebase = s * EPS + j * SCHUNK
        pltpu.sync_copy(m_hbm.at[c, pl.ds(ebase, SCHUNK)], data_v)
        pltpu.sync_copy(data_v, shared_sum.at[idx_v.at[j]], add=True)
        return carry

    lax.fori_loop(0, SCHUNKS, chunk, 0)

    plsc.subcore_barrier()

    def copy_out(t):
        rows = pl.ds(t * TASK, TASK)
        pltpu.sync_copy(shared_sum.at[rows], task_v)
        pltpu.sync_copy(task_v, sout_hbm.at[c, rows])

    def out_loop(k, carry):
        copy_out(s + k * NS)
        return carry

    lax.fori_loop(0, NTASK // NS, out_loop, 0)

    @pl.when(s < NTASK % NS)
    def _():
        copy_out((NTASK // NS) * NS + s)


def _sc_scatter(m, src3d, zsum):
    f = pl.kernel(
        _scatter_body,
        out_type=jax.ShapeDtypeStruct((2, N, DH), jnp.float32),
        mesh=_mesh(),
        scratch_types=[
            pltpu.VMEM((SCHUNKS, SCHUNK), jnp.int32),
            pltpu.VMEM((SCHUNK, DH), jnp.float32),
            pltpu.VMEM((TASK, DH), jnp.float32),
            pltpu.VMEM_SHARED((N, DH), jnp.float32),
            pltpu.SemaphoreType.DMA,
        ],
    )
    return f(m, src3d, zsum)


CW = 128
CCHUNK = 40
CEPS = E // NC // NS
CCHUNKS = CEPS // CCHUNK


def _count_body(src4d_hbm, zcnt_hbm, ones_hbm, cnt_hbm,
                idx_v, ones_v, ctask_v, shared_cnt, sem):
    c = lax.axis_index("c")
    s = lax.axis_index("s")

    pltpu.sync_copy(zcnt_hbm, ctask_v)
    pltpu.sync_copy(ones_hbm, ones_v)
    pltpu.sync_copy(src4d_hbm.at[c, s], idx_v)

    def init_loop(k, carry):
        pltpu.sync_copy(ctask_v, shared_cnt.at[pl.ds((s + k * NS) * TASK, TASK)])
        return carry

    lax.fori_loop(0, NTASK // NS, init_loop, 0)

    @pl.when(s < NTASK % NS)
    def _():
        pltpu.sync_copy(ctask_v,
                        shared_cnt.at[pl.ds(((NTASK // NS) * NS + s) * TASK, TASK)])

    plsc.subcore_barrier()

    def chunk(j, carry):
        pltpu.sync_copy(ones_v, shared_cnt.at[idx_v.at[j]], add=True)
        return carry

    lax.fori_loop(0, CCHUNKS, chunk, 0)

    plsc.subcore_barrier()

    def copy_out(t):
        rows = pl.ds(t * TASK, TASK)
        pltpu.sync_copy(shared_cnt.at[rows], ctask_v)
        pltpu.sync_copy(ctask_v, cnt_hbm.at[c, rows])

    def out_loop(k, carry):
        copy_out(s + k * NS)
        return carry

    lax.fori_loop(0, NTASK // NS, out_loop, 0)

    @pl.when(s < NTASK % NS)
    def _():
        copy_out((NTASK // NS) * NS + s)


def _sc_count(src4d, zcnt, ones):
    f = pl.kernel(
        _count_body,
        out_type=jax.ShapeDtypeStruct((NC, N, CW), jnp.float32),
        mesh=_mesh(),
        scratch_types=[
            pltpu.VMEM((CCHUNKS, CCHUNK), jnp.int32),
            pltpu.VMEM((CCHUNK, CW), jnp.float32),
            pltpu.VMEM((TASK, CW), jnp.float32),
            pltpu.VMEM_SHARED((N, CW), jnp.float32),
            pltpu.SemaphoreType.DMA,
        ],
    )
    return f(src4d, zcnt, ones)


def _node_update_body(hv_ref, s0_ref, s1_ref, cnt_ref, wi_ref, bi_ref,
                      wo_ref, bo_ref, g1_ref, bb1_ref, g2_ref, bb2_ref,
                      out_ref):
    def ln(x, g, b, eps=1e-5):
        m = jnp.mean(x, axis=-1, keepdims=True)
        v = jnp.mean((x - m) ** 2, axis=-1, keepdims=True)
        return (x - m) * lax.rsqrt(v + eps) * g + b

    sums = jnp.concatenate([s0_ref[...], s1_ref[...]], axis=-1)
    cnt = (cnt_ref[0] + cnt_ref[1])[:, :1]
    mean = sums / jnp.maximum(cnt, 1.0)
    h = ln(hv_ref[...] + mean * (1.0 / SCALE), g1_ref[...], bb1_ref[...])
    ffh = jnp.maximum(
        jnp.dot(h, wi_ref[...], preferred_element_type=jnp.float32)
        + bi_ref[...], 0.0)
    dh = jnp.dot(ffh, wo_ref[...], preferred_element_type=jnp.float32) + bo_ref[...]
    out_ref[...] = ln(h + dh, g2_ref[...], bb2_ref[...])


def _node_update(h_V, s0, s1, cnt, Wi, bi, Wo, bo, ln1_g, ln1_b, ln2_g, ln2_b,
                 blk=1000):
    grid = (N // blk,)
    row = lambda c: pl.BlockSpec((blk, c), lambda i: (i, 0))
    vec = lambda c: pl.BlockSpec((c,), lambda i: (0,))
    return pl.pallas_call(
        _node_update_body,
        grid=grid,
        in_specs=[
            row(D), row(DH), row(DH),
            pl.BlockSpec((2, blk, CW), lambda i: (0, i, 0)),
            pl.BlockSpec((D, FF), lambda i: (0, 0)), vec(FF),
            pl.BlockSpec((FF, D), lambda i: (0, 0)), vec(D),
            vec(D), vec(D), vec(D), vec(D),
        ],
        out_specs=row(D),
        out_shape=jax.ShapeDtypeStruct((N, D), jnp.float32),
    )(h_V, s0, s1, cnt, Wi, bi, Wo, bo, ln1_g, ln1_b, ln2_g, ln2_b)


def kernel(h_V, h_E, src_idx, batch_id, dst_idx, W1, b1, W2, b2, W3, b3,
           Wi, bi, Wo, bo, ln1_g, ln1_b, ln2_g, ln2_b):
    w_a = W1[:D]
    w_b = W1[D:2 * D]
    w_c = W1[2 * D:]
    w_ac = jnp.concatenate([w_a, w_c], axis=1)

    src3d = src_idx.reshape(NS, SCHUNKS, SCHUNK)

    a, c = _node_proj(h_V, w_ac)
    g1, g2 = _sc_gather(a, c, src_idx, dst_idx)
    m = _edge_mlp(h_E, g1, g2, w_b, b1, W2, b2, W3, b3)

    src4d = src_idx.reshape(NC, NS, CCHUNKS, CCHUNK)
    zsum = jnp.zeros((TASK, DH), jnp.float32)
    zcnt = jnp.zeros((TASK, CW), jnp.float32)
    ones = jnp.ones((CCHUNK, CW), jnp.float32)
    sums = _sc_scatter(m, src3d, zsum)
    s0, s1 = sums[0], sums[1]
    cnt = _sc_count(src4d, zcnt, ones)

    return _node_update(h_V, s0, s1, cnt, Wi, bi, Wo, bo,
                        ln1_g, ln1_b, ln2_g, ln2_b)


# speedup vs baseline: 3.5925x; 1.1816x over previous
import functools

import jax
import jax.numpy as jnp
from jax import lax
from jax.experimental import pallas as pl
from jax.experimental.pallas import tpu as pltpu
from jax.experimental.pallas import tpu_sc as plsc

N = 10000
E = 160000
D = 256
DH = D // 2
FF = 4 * D
SCALE = 30.0

NC = 2
NS = 16
NW = NC * NS
EPW = E // NW
GCHUNK = 128
GFULL = EPW // GCHUNK
GTAIL = EPW - GFULL * GCHUNK
EPS = E // NS
SCHUNK = 80
SCHUNKS = EPS // SCHUNK
TASK = 40
NTASK = N // TASK

@functools.cache
def _mesh():
    return plsc.VectorSubcoreMesh(core_axis_name="c", subcore_axis_name="s",
                                  num_cores=NC, num_subcores=NS)


def _gelu(x):
    return 0.5 * x * (1.0 + lax.erf(x * 0.7071067811865476))


def _node_proj_body(hv_ref, wac_ref, a_ref, c_ref):
    y = jnp.dot(hv_ref[...], wac_ref[...], preferred_element_type=jnp.float32)
    a_ref[...] = y[:, :D]
    c_ref[...] = y[:, D:]


def _node_proj(h_V, w_ac, blk=1000):
    grid = (N // blk,)
    return pl.pallas_call(
        _node_proj_body,
        grid=grid,
        in_specs=[
            pl.BlockSpec((blk, D), lambda i: (i, 0)),
            pl.BlockSpec((D, 2 * D), lambda i: (0, 0)),
        ],
        out_specs=[
            pl.BlockSpec((blk, D), lambda i: (i, 0)),
            pl.BlockSpec((blk, D), lambda i: (i, 0)),
        ],
        out_shape=[
            jax.ShapeDtypeStruct((N, D), jnp.float32),
            jax.ShapeDtypeStruct((N, D), jnp.float32),
        ],
    )(h_V, w_ac)


def _gather_body(a_hbm, c_hbm, src_hbm, dst_hbm, g1_hbm, g2_hbm,
                 sidx_v, didx_v, bufa, bufc, sema, semc):
    c = lax.axis_index("c")
    s = lax.axis_index("s")
    wid = s * NC + c
    base = wid * EPW
    pltpu.sync_copy(src_hbm.at[pl.ds(base, EPW)], sidx_v)
    pltpu.sync_copy(dst_hbm.at[pl.ds(base, EPW)], didx_v)

    pltpu.async_copy(a_hbm.at[sidx_v.at[pl.ds(0, GCHUNK)]], bufa, sema)
    pltpu.async_copy(c_hbm.at[didx_v.at[pl.ds(0, GCHUNK)]], bufc, semc)

    def chunk(j, carry):
        off = j * GCHUNK
        pltpu.make_async_copy(a_hbm.at[pl.ds(0, GCHUNK)], bufa, sema).wait()
        pltpu.sync_copy(bufa, g1_hbm.at[pl.ds(base + off, GCHUNK)])

        @pl.when(j + 1 < GFULL)
        def _():
            pltpu.async_copy(
                a_hbm.at[sidx_v.at[pl.ds(off + GCHUNK, GCHUNK)]], bufa, sema)

        pltpu.make_async_copy(c_hbm.at[pl.ds(0, GCHUNK)], bufc, semc).wait()
        pltpu.sync_copy(bufc, g2_hbm.at[pl.ds(base + off, GCHUNK)])

        @pl.when(j + 1 < GFULL)
        def _():
            pltpu.async_copy(
                c_hbm.at[didx_v.at[pl.ds(off + GCHUNK, GCHUNK)]], bufc, semc)

        return carry

    lax.fori_loop(0, GFULL, chunk, 0)
    toff = GFULL * GCHUNK
    ta = bufa.at[pl.ds(0, GTAIL)]
    tc = bufc.at[pl.ds(0, GTAIL)]
    pltpu.async_copy(a_hbm.at[sidx_v.at[pl.ds(toff, GTAIL)]], ta, sema).wait()
    pltpu.sync_copy(ta, g1_hbm.at[pl.ds(base + toff, GTAIL)])
    pltpu.async_copy(c_hbm.at[didx_v.at[pl.ds(toff, GTAIL)]], tc, semc).wait()
    pltpu.sync_copy(tc, g2_hbm.at[pl.ds(base + toff, GTAIL)])


def _sc_gather(a, c, src_idx, dst_idx):
    f = pl.kernel(
        _gather_body,
        out_type=[
            jax.ShapeDtypeStruct((E, D), jnp.float32),
            jax.ShapeDtypeStruct((E, D), jnp.float32),
        ],
        mesh=_mesh(),
        scratch_types=[
            pltpu.VMEM((EPW,), jnp.int32),
            pltpu.VMEM((EPW,), jnp.int32),
            pltpu.VMEM((GCHUNK, D), jnp.float32),
            pltpu.VMEM((GCHUNK, D), jnp.float32),
            pltpu.SemaphoreType.DMA,
            pltpu.SemaphoreType.DMA,
        ],
    )
    return f(a, c, src_idx, dst_idx)


def _edge_mlp_body(he_ref, g1_ref, g2_ref, w1b_ref, b1_ref, w2_ref, b2_ref,
                   w3_ref, b3_ref, m_ref):
    x = (g1_ref[...] + g2_ref[...] + b1_ref[...]
         + jnp.dot(he_ref[...], w1b_ref[...], preferred_element_type=jnp.float32))
    x = _gelu(x)
    y = _gelu(jnp.dot(x, w2_ref[...], preferred_element_type=jnp.float32)
              + b2_ref[...])
    m = jnp.dot(y, w3_ref[...], preferred_element_type=jnp.float32) + b3_ref[...]
    m_ref[0] = m[:, :DH]
    m_ref[1] = m[:, DH:]


def _edge_mlp(h_E, g1, g2, w1b, b1, w2, b2, w3, b3, blk=1000):
    grid = (E // blk,)
    full = lambda r, c: pl.BlockSpec((r, c), lambda i: (0, 0))
    row = lambda c: pl.BlockSpec((blk, c), lambda i: (i, 0))
    return pl.pallas_call(
        _edge_mlp_body,
        grid=grid,
        in_specs=[
            row(D), row(D), row(D),
            full(D, D),
            pl.BlockSpec((D,), lambda i: (0,)),
            full(D, D),
            pl.BlockSpec((D,), lambda i: (0,)),
            full(D, D),
            pl.BlockSpec((D,), lambda i: (0,)),
        ],
        out_specs=pl.BlockSpec((2, blk, DH), lambda i: (0, i, 0)),
        out_shape=jax.ShapeDtypeStruct((2, E, DH), jnp.float32),
    )(h_E, g1, g2, w1b, b1, w2, b2, w3, b3)


def _scatter_body(m_hbm, src3d_hbm, zsum_hbm, sout_hbm,
                  idx_v, data_v, data2_v, task_v, shared_sum, sem, sem2):
    c = lax.axis_index("c")
    s = lax.axis_index("s")

    pltpu.sync_copy(zsum_hbm, task_v)
    pltpu.sync_copy(src3d_hbm.at[s], idx_v)

    def init_loop(k, carry):
        pltpu.sync_copy(task_v, shared_sum.at[pl.ds((s + k * NS) * TASK, TASK)])
        return carry

    lax.fori_loop(0, NTASK // NS, init_loop, 0)

    @pl.when(s < NTASK % NS)
    def _():
        pltpu.sync_copy(task_v,
                        shared_sum.at[pl.ds(((NTASK // NS) * NS + s) * TASK, TASK)])

    plsc.subcore_barrier()

    ebase0 = s * EPS
    pltpu.async_copy(m_hbm.at[c, pl.ds(ebase0, SCHUNK)], data_v, sem)
    pltpu.async_copy(m_hbm.at[c, pl.ds(ebase0 + SCHUNK, SCHUNK)], data2_v, sem2)

    def step(j, buf, bsem):
        pltpu.make_async_copy(m_hbm.at[c, pl.ds(0, SCHUNK)], buf, bsem).wait()
        pltpu.sync_copy(buf, shared_sum.at[idx_v.at[j]], add=True)

        @pl.when(j + 2 < SCHUNKS)
        def _():
            pltpu.async_copy(
                m_hbm.at[c, pl.ds(ebase0 + (j + 2) * SCHUNK, SCHUNK)], buf, bsem)

    def chunk(jj, carry):
        step(2 * jj, data_v, sem)
        step(2 * jj + 1, data2_v, sem2)
        return carry

    lax.fori_loop(0, SCHUNKS // 2, chunk, 0)
    step(SCHUNKS - 1, data_v, sem)

    plsc.subcore_barrier()

    def copy_out(t):
        rows = pl.ds(t * TASK, TASK)
        pltpu.sync_copy(shared_sum.at[rows], task_v)
        pltpu.sync_copy(task_v, sout_hbm.at[c, rows])

    def out_loop(k, carry):
        copy_out(s + k * NS)
        return carry

    lax.fori_loop(0, NTASK // NS, out_loop, 0)

    @pl.when(s < NTASK % NS)
    def _():
        copy_out((NTASK // NS) * NS + s)


def _sc_scatter(m, src3d, zsum):
    f = pl.kernel(
        _scatter_body,
        out_type=jax.ShapeDtypeStruct((2, N, DH), jnp.float32),
        mesh=_mesh(),
        scratch_types=[
            pltpu.VMEM((SCHUNKS, SCHUNK), jnp.int32),
            pltpu.VMEM((SCHUNK, DH), jnp.float32),
            pltpu.VMEM((SCHUNK, DH), jnp.float32),
            pltpu.VMEM((TASK, DH), jnp.float32),
            pltpu.VMEM_SHARED((N, DH), jnp.float32),
            pltpu.SemaphoreType.DMA,
            pltpu.SemaphoreType.DMA,
        ],
    )
    return f(m, src3d, zsum)


CW = 128
CCHUNK = 40
CEPS = E // NC // NS
CCHUNKS = CEPS // CCHUNK


def _count_body(src4d_hbm, zcnt_hbm, ones_hbm, cnt_hbm,
                idx_v, ones_v, ctask_v, shared_cnt, sem):
    c = lax.axis_index("c")
    s = lax.axis_index("s")

    pltpu.sync_copy(zcnt_hbm, ctask_v)
    pltpu.sync_copy(ones_hbm, ones_v)
    pltpu.sync_copy(src4d_hbm.at[c, s], idx_v)

    def init_loop(k, carry):
        pltpu.sync_copy(ctask_v, shared_cnt.at[pl.ds((s + k * NS) * TASK, TASK)])
        return carry

    lax.fori_loop(0, NTASK // NS, init_loop, 0)

    @pl.when(s < NTASK % NS)
    def _():
        pltpu.sync_copy(ctask_v,
                        shared_cnt.at[pl.ds(((NTASK // NS) * NS + s) * TASK, TASK)])

    plsc.subcore_barrier()

    def chunk(j, carry):
        pltpu.sync_copy(ones_v, shared_cnt.at[idx_v.at[j]], add=True)
        return carry

    lax.fori_loop(0, CCHUNKS, chunk, 0)

    plsc.subcore_barrier()

    def copy_out(t):
        rows = pl.ds(t * TASK, TASK)
        pltpu.sync_copy(shared_cnt.at[rows], ctask_v)
        pltpu.sync_copy(ctask_v, cnt_hbm.at[c, rows])

    def out_loop(k, carry):
        copy_out(s + k * NS)
        return carry

    lax.fori_loop(0, NTASK // NS, out_loop, 0)

    @pl.when(s < NTASK % NS)
    def _():
        copy_out((NTASK // NS) * NS + s)


def _sc_count(src4d, zcnt, ones):
    f = pl.kernel(
        _count_body,
        out_type=jax.ShapeDtypeStruct((NC, N, CW), jnp.float32),
        mesh=_mesh(),
        scratch_types=[
            pltpu.VMEM((CCHUNKS, CCHUNK), jnp.int32),
            pltpu.VMEM((CCHUNK, CW), jnp.float32),
            pltpu.VMEM((TASK, CW), jnp.float32),
            pltpu.VMEM_SHARED((N, CW), jnp.float32),
            pltpu.SemaphoreType.DMA,
        ],
    )
    return f(src4d, zcnt, ones)


def _node_update_body(hv_ref, s0_ref, s1_ref, cnt_ref, wi_ref, bi_ref,
                      wo_ref, bo_ref, g1_ref, bb1_ref, g2_ref, bb2_ref,
                      out_ref):
    def ln(x, g, b, eps=1e-5):
        m = jnp.mean(x, axis=-1, keepdims=True)
        v = jnp.mean((x - m) ** 2, axis=-1, keepdims=True)
        return (x - m) * lax.rsqrt(v + eps) * g + b

    sums = jnp.concatenate([s0_ref[...], s1_ref[...]], axis=-1)
    cnt = (cnt_ref[0] + cnt_ref[1])[:, :1]
    mean = sums / jnp.maximum(cnt, 1.0)
    h = ln(hv_ref[...] + mean * (1.0 / SCALE), g1_ref[...], bb1_ref[...])
    ffh = jnp.maximum(
        jnp.dot(h, wi_ref[...], preferred_element_type=jnp.float32)
        + bi_ref[...], 0.0)
    dh = jnp.dot(ffh, wo_ref[...], preferred_element_type=jnp.float32) + bo_ref[...]
    out_ref[...] = ln(h + dh, g2_ref[...], bb2_ref[...])


def _node_update(h_V, s0, s1, cnt, Wi, bi, Wo, bo, ln1_g, ln1_b, ln2_g, ln2_b,
                 blk=1000):
    grid = (N // blk,)
    row = lambda c: pl.BlockSpec((blk, c), lambda i: (i, 0))
    vec = lambda c: pl.BlockSpec((c,), lambda i: (0,))
    return pl.pallas_call(
        _node_update_body,
        grid=grid,
        in_specs=[
            row(D), row(DH), row(DH),
            pl.BlockSpec((2, blk, CW), lambda i: (0, i, 0)),
            pl.BlockSpec((D, FF), lambda i: (0, 0)), vec(FF),
            pl.BlockSpec((FF, D), lambda i: (0, 0)), vec(D),
            vec(D), vec(D), vec(D), vec(D),
        ],
        out_specs=row(D),
        out_shape=jax.ShapeDtypeStruct((N, D), jnp.float32),
    )(h_V, s0, s1, cnt, Wi, bi, Wo, bo, ln1_g, ln1_b, ln2_g, ln2_b)


def kernel(h_V, h_E, src_idx, batch_id, dst_idx, W1, b1, W2, b2, W3, b3,
           Wi, bi, Wo, bo, ln1_g, ln1_b, ln2_g, ln2_b):
    w_a = W1[:D]
    w_b = W1[D:2 * D]
    w_c = W1[2 * D:]
    w_ac = jnp.concatenate([w_a, w_c], axis=1)

    src3d = src_idx.reshape(NS, SCHUNKS, SCHUNK)

    a, c = _node_proj(h_V, w_ac)
    g1, g2 = _sc_gather(a, c, src_idx, dst_idx)
    m = _edge_mlp(h_E, g1, g2, w_b, b1, W2, b2, W3, b3)

    src4d = src_idx.reshape(NC, NS, CCHUNKS, CCHUNK)
    zsum = jnp.zeros((TASK, DH), jnp.float32)
    zcnt = jnp.zeros((TASK, CW), jnp.float32)
    ones = jnp.ones((CCHUNK, CW), jnp.float32)
    sums = _sc_scatter(m, src3d, zsum)
    s0, s1 = sums[0], sums[1]
    cnt = _sc_count(src4d, zcnt, ones)

    return _node_update(h_V, s0, s1, cnt, Wi, bi, Wo, bo,
                        ln1_g, ln1_b, ln2_g, ln2_b)


# speedup vs baseline: 3.9019x; 1.0861x over previous
import functools

import jax
import jax.numpy as jnp
from jax import lax
from jax.experimental import pallas as pl
from jax.experimental.pallas import tpu as pltpu
from jax.experimental.pallas import tpu_sc as plsc

N = 10000
E = 160000
D = 256
DH = D // 2
FF = 4 * D
SCALE = 30.0

NC = 2
NS = 16
NW = NC * NS
EPW = E // NW
GCHUNK = 128
GFULL = EPW // GCHUNK
GTAIL = EPW - GFULL * GCHUNK
EPS = E // NS
SCHUNK = 80
SCHUNKS = EPS // SCHUNK
TASK = 40
NTASK = N // TASK

@functools.cache
def _mesh():
    return plsc.VectorSubcoreMesh(core_axis_name="c", subcore_axis_name="s",
                                  num_cores=NC, num_subcores=NS)


def _gelu(x):
    return 0.5 * x * (1.0 + lax.erf(x * 0.7071067811865476))


def _node_proj_body(hv_ref, wac_ref, a_ref, c_ref):
    y = jnp.dot(hv_ref[...], wac_ref[...], preferred_element_type=jnp.float32)
    a_ref[...] = y[:, :D]
    c_ref[...] = y[:, D:]


def _node_proj(h_V, w_ac, blk=1000):
    grid = (N // blk,)
    return pl.pallas_call(
        _node_proj_body,
        grid=grid,
        in_specs=[
            pl.BlockSpec((blk, D), lambda i: (i, 0)),
            pl.BlockSpec((D, 2 * D), lambda i: (0, 0)),
        ],
        out_specs=[
            pl.BlockSpec((blk, D), lambda i: (i, 0)),
            pl.BlockSpec((blk, D), lambda i: (i, 0)),
        ],
        out_shape=[
            jax.ShapeDtypeStruct((N, D), jnp.float32),
            jax.ShapeDtypeStruct((N, D), jnp.float32),
        ],
    )(h_V, w_ac)


def _gather_body(a_hbm, c_hbm, src_hbm, dst_hbm, g1_hbm, g2_hbm,
                 sidx_v, didx_v, bufa, bufc, sema, semc):
    c = lax.axis_index("c")
    s = lax.axis_index("s")
    wid = s * NC + c
    base = wid * EPW
    pltpu.sync_copy(src_hbm.at[pl.ds(base, EPW)], sidx_v)
    pltpu.sync_copy(dst_hbm.at[pl.ds(base, EPW)], didx_v)

    pltpu.async_copy(a_hbm.at[sidx_v.at[pl.ds(0, GCHUNK)]], bufa, sema)
    pltpu.async_copy(c_hbm.at[didx_v.at[pl.ds(0, GCHUNK)]], bufc, semc)

    def chunk(j, carry):
        off = j * GCHUNK
        pltpu.make_async_copy(a_hbm.at[pl.ds(0, GCHUNK)], bufa, sema).wait()
        pltpu.sync_copy(bufa, g1_hbm.at[pl.ds(base + off, GCHUNK)])

        @pl.when(j + 1 < GFULL)
        def _():
            pltpu.async_copy(
                a_hbm.at[sidx_v.at[pl.ds(off + GCHUNK, GCHUNK)]], bufa, sema)

        pltpu.make_async_copy(c_hbm.at[pl.ds(0, GCHUNK)], bufc, semc).wait()
        pltpu.sync_copy(bufc, g2_hbm.at[pl.ds(base + off, GCHUNK)])

        @pl.when(j + 1 < GFULL)
        def _():
            pltpu.async_copy(
                c_hbm.at[didx_v.at[pl.ds(off + GCHUNK, GCHUNK)]], bufc, semc)

        return carry

    lax.fori_loop(0, GFULL, chunk, 0)
    toff = GFULL * GCHUNK
    ta = bufa.at[pl.ds(0, GTAIL)]
    tc = bufc.at[pl.ds(0, GTAIL)]
    pltpu.async_copy(a_hbm.at[sidx_v.at[pl.ds(toff, GTAIL)]], ta, sema).wait()
    pltpu.sync_copy(ta, g1_hbm.at[pl.ds(base + toff, GTAIL)])
    pltpu.async_copy(c_hbm.at[didx_v.at[pl.ds(toff, GTAIL)]], tc, semc).wait()
    pltpu.sync_copy(tc, g2_hbm.at[pl.ds(base + toff, GTAIL)])


def _sc_gather(a, c, src_idx, dst_idx):
    f = pl.kernel(
        _gather_body,
        out_type=[
            jax.ShapeDtypeStruct((E, D), jnp.float32),
            jax.ShapeDtypeStruct((E, D), jnp.float32),
        ],
        mesh=_mesh(),
        scratch_types=[
            pltpu.VMEM((EPW,), jnp.int32),
            pltpu.VMEM((EPW,), jnp.int32),
            pltpu.VMEM((GCHUNK, D), jnp.float32),
            pltpu.VMEM((GCHUNK, D), jnp.float32),
            pltpu.SemaphoreType.DMA,
            pltpu.SemaphoreType.DMA,
        ],
    )
    return f(a, c, src_idx, dst_idx)


def _edge_mlp_body(he_ref, g1_ref, g2_ref, w1b_ref, b1_ref, w2_ref, b2_ref,
                   w3_ref, b3_ref, m_ref):
    x = (g1_ref[...] + g2_ref[...] + b1_ref[...]
         + jnp.dot(he_ref[...], w1b_ref[...], preferred_element_type=jnp.float32))
    x = _gelu(x)
    y = _gelu(jnp.dot(x, w2_ref[...], preferred_element_type=jnp.float32)
              + b2_ref[...])
    m = jnp.dot(y, w3_ref[...], preferred_element_type=jnp.float32) + b3_ref[...]
    m_ref[0] = m[:, :DH]
    m_ref[1] = m[:, DH:]


def _edge_mlp(h_E, g1, g2, w1b, b1, w2, b2, w3, b3, blk=2000):
    grid = (E // blk,)
    full = lambda r, c: pl.BlockSpec((r, c), lambda i: (0, 0))
    row = lambda c: pl.BlockSpec((blk, c), lambda i: (i, 0))
    return pl.pallas_call(
        _edge_mlp_body,
        grid=grid,
        in_specs=[
            row(D), row(D), row(D),
            full(D, D),
            pl.BlockSpec((D,), lambda i: (0,)),
            full(D, D),
            pl.BlockSpec((D,), lambda i: (0,)),
            full(D, D),
            pl.BlockSpec((D,), lambda i: (0,)),
        ],
        out_specs=pl.BlockSpec((2, blk, DH), lambda i: (0, i, 0)),
        out_shape=jax.ShapeDtypeStruct((2, E, DH), jnp.float32),
    )(h_E, g1, g2, w1b, b1, w2, b2, w3, b3)


def _scatter_body(m_hbm, src3d_hbm, zsum_hbm, sout_hbm,
                  idx_v, data_v, data2_v, task_v, shared_sum, sem, sem2):
    c = lax.axis_index("c")
    s = lax.axis_index("s")

    pltpu.sync_copy(zsum_hbm, task_v)
    pltpu.sync_copy(src3d_hbm.at[s], idx_v)

    def init_loop(k, carry):
        pltpu.sync_copy(task_v, shared_sum.at[pl.ds((s + k * NS) * TASK, TASK)])
        return carry

    lax.fori_loop(0, NTASK // NS, init_loop, 0)

    @pl.when(s < NTASK % NS)
    def _():
        pltpu.sync_copy(task_v,
                        shared_sum.at[pl.ds(((NTASK // NS) * NS + s) * TASK, TASK)])

    plsc.subcore_barrier()

    ebase0 = s * EPS
    pltpu.async_copy(m_hbm.at[c, pl.ds(ebase0, SCHUNK)], data_v, sem)
    pltpu.async_copy(m_hbm.at[c, pl.ds(ebase0 + SCHUNK, SCHUNK)], data2_v, sem2)

    def step(j, buf, bsem):
        pltpu.make_async_copy(m_hbm.at[c, pl.ds(0, SCHUNK)], buf, bsem).wait()
        pltpu.sync_copy(buf, shared_sum.at[idx_v.at[j]], add=True)

        @pl.when(j + 2 < SCHUNKS)
        def _():
            pltpu.async_copy(
                m_hbm.at[c, pl.ds(ebase0 + (j + 2) * SCHUNK, SCHUNK)], buf, bsem)

    def chunk(jj, carry):
        step(2 * jj, data_v, sem)
        step(2 * jj + 1, data2_v, sem2)
        return carry

    lax.fori_loop(0, SCHUNKS // 2, chunk, 0)
    step(SCHUNKS - 1, data_v, sem)

    plsc.subcore_barrier()

    def copy_out(t):
        rows = pl.ds(t * TASK, TASK)
        pltpu.sync_copy(shared_sum.at[rows], task_v)
        pltpu.sync_copy(task_v, sout_hbm.at[c, rows])

    def out_loop(k, carry):
        copy_out(s + k * NS)
        return carry

    lax.fori_loop(0, NTASK // NS, out_loop, 0)

    @pl.when(s < NTASK % NS)
    def _():
        copy_out((NTASK // NS) * NS + s)


def _sc_scatter(m, src3d, zsum):
    f = pl.kernel(
        _scatter_body,
        out_type=jax.ShapeDtypeStruct((2, N, DH), jnp.float32),
        mesh=_mesh(),
        scratch_types=[
            pltpu.VMEM((SCHUNKS, SCHUNK), jnp.int32),
            pltpu.VMEM((SCHUNK, DH), jnp.float32),
            pltpu.VMEM((SCHUNK, DH), jnp.float32),
            pltpu.VMEM((TASK, DH), jnp.float32),
            pltpu.VMEM_SHARED((N, DH), jnp.float32),
            pltpu.SemaphoreType.DMA,
            pltpu.SemaphoreType.DMA,
        ],
    )
    return f(m, src3d, zsum)


CW = 128
CCHUNK = 40
CEPS = E // NC // NS
CCHUNKS = CEPS // CCHUNK


def _count_body(src4d_hbm, zcnt_hbm, ones_hbm, cnt_hbm,
                idx_v, ones_v, ctask_v, shared_cnt, sem):
    c = lax.axis_index("c")
    s = lax.axis_index("s")

    pltpu.sync_copy(zcnt_hbm, ctask_v)
    pltpu.sync_copy(ones_hbm, ones_v)
    pltpu.sync_copy(src4d_hbm.at[c, s], idx_v)

    def init_loop(k, carry):
        pltpu.sync_copy(ctask_v, shared_cnt.at[pl.ds((s + k * NS) * TASK, TASK)])
        return carry

    lax.fori_loop(0, NTASK // NS, init_loop, 0)

    @pl.when(s < NTASK % NS)
    def _():
        pltpu.sync_copy(ctask_v,
                        shared_cnt.at[pl.ds(((NTASK // NS) * NS + s) * TASK, TASK)])

    plsc.subcore_barrier()

    def chunk(j, carry):
        pltpu.sync_copy(ones_v, shared_cnt.at[idx_v.at[j]], add=True)
        return carry

    lax.fori_loop(0, CCHUNKS, chunk, 0)

    plsc.subcore_barrier()

    def copy_out(t):
        rows = pl.ds(t * TASK, TASK)
        pltpu.sync_copy(shared_cnt.at[rows], ctask_v)
        pltpu.sync_copy(ctask_v, cnt_hbm.at[c, rows])

    def out_loop(k, carry):
        copy_out(s + k * NS)
        return carry

    lax.fori_loop(0, NTASK // NS, out_loop, 0)

    @pl.when(s < NTASK % NS)
    def _():
        copy_out((NTASK // NS) * NS + s)


def _sc_count(src4d, zcnt, ones):
    f = pl.kernel(
        _count_body,
        out_type=jax.ShapeDtypeStruct((NC, N, CW), jnp.float32),
        mesh=_mesh(),
        scratch_types=[
            pltpu.VMEM((CCHUNKS, CCHUNK), jnp.int32),
            pltpu.VMEM((CCHUNK, CW), jnp.float32),
            pltpu.VMEM((TASK, CW), jnp.float32),
            pltpu.VMEM_SHARED((N, CW), jnp.float32),
            pltpu.SemaphoreType.DMA,
        ],
    )
    return f(src4d, zcnt, ones)


def _node_update_body(hv_ref, s0_ref, s1_ref, cnt_ref, wi_ref, bi_ref,
                      wo_ref, bo_ref, g1_ref, bb1_ref, g2_ref, bb2_ref,
                      out_ref):
    def ln(x, g, b, eps=1e-5):
        m = jnp.mean(x, axis=-1, keepdims=True)
        v = jnp.mean((x - m) ** 2, axis=-1, keepdims=True)
        return (x - m) * lax.rsqrt(v + eps) * g + b

    sums = jnp.concatenate([s0_ref[...], s1_ref[...]], axis=-1)
    cnt = (cnt_ref[0] + cnt_ref[1])[:, :1]
    mean = sums / jnp.maximum(cnt, 1.0)
    h = ln(hv_ref[...] + mean * (1.0 / SCALE), g1_ref[...], bb1_ref[...])
    ffh = jnp.maximum(
        jnp.dot(h, wi_ref[...], preferred_element_type=jnp.float32)
        + bi_ref[...], 0.0)
    dh = jnp.dot(ffh, wo_ref[...], preferred_element_type=jnp.float32) + bo_ref[...]
    out_ref[...] = ln(h + dh, g2_ref[...], bb2_ref[...])


def _node_update(h_V, s0, s1, cnt, Wi, bi, Wo, bo, ln1_g, ln1_b, ln2_g, ln2_b,
                 blk=2000):
    grid = (N // blk,)
    row = lambda c: pl.BlockSpec((blk, c), lambda i: (i, 0))
    vec = lambda c: pl.BlockSpec((c,), lambda i: (0,))
    return pl.pallas_call(
        _node_update_body,
        grid=grid,
        in_specs=[
            row(D), row(DH), row(DH),
            pl.BlockSpec((2, blk, CW), lambda i: (0, i, 0)),
            pl.BlockSpec((D, FF), lambda i: (0, 0)), vec(FF),
            pl.BlockSpec((FF, D), lambda i: (0, 0)), vec(D),
            vec(D), vec(D), vec(D), vec(D),
        ],
        out_specs=row(D),
        out_shape=jax.ShapeDtypeStruct((N, D), jnp.float32),
    )(h_V, s0, s1, cnt, Wi, bi, Wo, bo, ln1_g, ln1_b, ln2_g, ln2_b)


def kernel(h_V, h_E, src_idx, batch_id, dst_idx, W1, b1, W2, b2, W3, b3,
           Wi, bi, Wo, bo, ln1_g, ln1_b, ln2_g, ln2_b):
    w_a = W1[:D]
    w_b = W1[D:2 * D]
    w_c = W1[2 * D:]
    w_ac = jnp.concatenate([w_a, w_c], axis=1)

    src3d = src_idx.reshape(NS, SCHUNKS, SCHUNK)

    a, c = _node_proj(h_V, w_ac)
    g1, g2 = _sc_gather(a, c, src_idx, dst_idx)
    m = _edge_mlp(h_E, g1, g2, w_b, b1, W2, b2, W3, b3)

    src4d = src_idx.reshape(NC, NS, CCHUNKS, CCHUNK)
    zsum = jnp.zeros((TASK, DH), jnp.float32)
    zcnt = jnp.zeros((TASK, CW), jnp.float32)
    ones = jnp.ones((CCHUNK, CW), jnp.float32)
    sums = _sc_scatter(m, src3d, zsum)
    s0, s1 = sums[0], sums[1]
    cnt = _sc_count(src4d, zcnt, ones)

    return _node_update(h_V, s0, s1, cnt, Wi, bi, Wo, bo,
                        ln1_g, ln1_b, ln2_g, ln2_b)
